# baseline (speedup 1.0000x reference)
import functools

import jax
import jax.numpy as jnp
from jax import lax
from jax.experimental import pallas as pl
from jax.experimental.pallas import tpu as pltpu
from jax.experimental.pallas import tpu_sc as plsc

N_NODES = 10000
N_EDGES = 320000
HID = 128
N_STEPS = 3

ROW_BLK = 1000
GRID = N_NODES // ROW_BLK

NC = 2
NS = 16
NW = NC * NS
CHUNK = 128
TILE_CHUNKS = 80
HALF = TILE_CHUNKS // 2
E_PAD = NW * TILE_CHUNKS * CHUNK
NBUF = 2
N_PAD = 10240
ROWS_PER_TILE = N_PAD // NS

def _dot(a, b):
    return jnp.dot(a.astype(jnp.bfloat16), b.astype(jnp.bfloat16),
                   preferred_element_type=jnp.float32)


def _embed_body(x_ref, wembt, bemb, wgt, bg, h_ref, m_ref):
    x = x_ref[...]
    h = _dot(x, wembt[...]) + bemb[...]
    h_ref[...] = h
    m_ref[...] = _dot(h, wgt[...]) + bg[...]


def _gru_core(a0_ref, a1_ref, h_ref, wiht, bih, whht, bhh):
    a = a0_ref[0] + a1_ref[0]
    h = h_ref[...]
    gi = _dot(a, wiht[...]) + bih[...]
    gh = _dot(h, whht[...]) + bhh[...]
    r = jax.nn.sigmoid(gi[:, :HID] + gh[:, :HID])
    z = jax.nn.sigmoid(gi[:, HID:2 * HID] + gh[:, HID:2 * HID])
    n = jnp.tanh(gi[:, 2 * HID:] + r * gh[:, 2 * HID:])
    return (1.0 - z) * n + z * h


def _gru_mid_body(a0_ref, a1_ref, h_ref, wiht, bih, whht, bhh, wgt, bg,
                  ho_ref, mo_ref):
    hn = _gru_core(a0_ref, a1_ref, h_ref, wiht, bih, whht, bhh)
    ho_ref[...] = hn
    mo_ref[...] = _dot(hn, wgt[...]) + bg[...]


def _gru_last_body(a0_ref, a1_ref, h_ref, wiht, bih, whht, bhh, woutt, bout,
                   out_ref):
    hn = _gru_core(a0_ref, a1_ref, h_ref, wiht, bih, whht, bhh)
    out_ref[...] = jnp.tanh(_dot(hn, woutt[...]) + bout[...])


def _row_spec(cols):
    return pl.BlockSpec((ROW_BLK, cols), lambda i: (i, 0))


def _full_spec(r, c):
    return pl.BlockSpec((r, c), lambda i: (0, 0))


def _part_spec(which):
    return pl.BlockSpec((1, ROW_BLK, HID), lambda i, w=which: (w, i, 0))


_f32 = jnp.float32


def _embed_call(x, wembt, bemb, wgt, bg):
    return pl.pallas_call(
        _embed_body,
        grid=(GRID,),
        in_specs=[
            _row_spec(HID),
            _full_spec(HID, HID), _full_spec(1, HID),
            _full_spec(HID, HID), _full_spec(1, HID),
        ],
        out_specs=[_row_spec(HID), _row_spec(HID)],
        out_shape=[
            jax.ShapeDtypeStruct((N_NODES, HID), _f32),
            jax.ShapeDtypeStruct((N_NODES, HID), _f32),
        ],
    )(x, wembt, bemb, wgt, bg)


def _gru_mid_call(parts, h, wiht, bih, whht, bhh, wgt, bg):
    return pl.pallas_call(
        _gru_mid_body,
        grid=(GRID,),
        in_specs=[
            _part_spec(0), _part_spec(1),
            _row_spec(HID),
            _full_spec(HID, 3 * HID), _full_spec(1, 3 * HID),
            _full_spec(HID, 3 * HID), _full_spec(1, 3 * HID),
            _full_spec(HID, HID), _full_spec(1, HID),
        ],
        out_specs=[_row_spec(HID), _row_spec(HID)],
        out_shape=[
            jax.ShapeDtypeStruct((N_NODES, HID), _f32),
            jax.ShapeDtypeStruct((N_NODES, HID), _f32),
        ],
    )(parts, parts, h, wiht, bih, whht, bhh, wgt, bg)


def _gru_last_call(parts, h, wiht, bih, whht, bhh, woutt, bout):
    return pl.pallas_call(
        _gru_last_body,
        grid=(GRID,),
        in_specs=[
            _part_spec(0), _part_spec(1),
            _row_spec(HID),
            _full_spec(HID, 3 * HID), _full_spec(1, 3 * HID),
            _full_spec(HID, 3 * HID), _full_spec(1, 3 * HID),
            _full_spec(HID, HID), _full_spec(1, HID),
        ],
        out_specs=_row_spec(HID),
        out_shape=jax.ShapeDtypeStruct((N_NODES, HID), _f32),
    )(parts, parts, h, wiht, bih, whht, bhh, woutt, bout)


@functools.partial(
    pl.kernel,
    out_type=jax.ShapeDtypeStruct((NC, N_PAD, HID), _f32),
    mesh=plsc.VectorSubcoreMesh(core_axis_name="c", subcore_axis_name="s"),
    scratch_types=[
        pltpu.VMEM((HALF, CHUNK), jnp.int32),
        pltpu.VMEM((HALF, CHUNK), jnp.int32),
        pltpu.VMEM((NBUF, CHUNK, HID), _f32),
        pltpu.VMEM_SHARED((N_PAD, HID), _f32),
        pltpu.SemaphoreType.DMA((NBUF,)),
        pltpu.SemaphoreType.DMA((NBUF,)),
    ],
)
def _msg_pass(m_hbm, srcs_hbm, dsts_hbm, zeros_hbm, out_hbm,
              src_t, dst_t, rows_v, a_sh, gsem, ssem):
    c = lax.axis_index("c")
    s = lax.axis_index("s")
    wid = c * NS + s

    def gather_start(cc, b):
        pltpu.async_copy(m_hbm.at[src_t.at[cc]], rows_v.at[b], gsem.at[b])

    def gather_wait(cc, b):
        pltpu.make_async_copy(m_hbm.at[src_t.at[cc]], rows_v.at[b],
                              gsem.at[b]).wait()

    def scatter_start(cc, b):
        pltpu.async_copy(rows_v.at[b], a_sh.at[dst_t.at[cc]], ssem.at[b],
                        add=True)

    def scatter_wait(cc, b):
        pltpu.make_async_copy(rows_v.at[b], a_sh.at[dst_t.at[cc]],
                              ssem.at[b]).wait()

    pltpu.sync_copy(srcs_hbm.at[wid, pl.ds(0, HALF)], src_t)
    pltpu.sync_copy(dsts_hbm.at[wid, pl.ds(0, HALF)], dst_t)
    gather_start(0, 0)
    pltpu.sync_copy(zeros_hbm.at[pl.ds(s * ROWS_PER_TILE, ROWS_PER_TILE)],
                    a_sh.at[pl.ds(s * ROWS_PER_TILE, ROWS_PER_TILE)])
    plsc.subcore_barrier()

    for hh in range(TILE_CHUNKS // HALF):
        if hh > 0:
            scatter_wait(HALF - 1, (HALF - 1) % NBUF)
            pltpu.sync_copy(srcs_hbm.at[wid, pl.ds(hh * HALF, HALF)], src_t)
            pltpu.sync_copy(dsts_hbm.at[wid, pl.ds(hh * HALF, HALF)], dst_t)
            gather_start(0, 0)

        @pl.loop(0, HALF, step=NBUF)
        def _(t):
            for b in range(NBUF):
                cc = t + b
                gather_wait(cc, b)
                scatter_start(cc, b)
                ob = 1 - b

                @pl.when(cc >= 1)
                def _():
                    scatter_wait(cc - 1, ob)

                @pl.when(cc + 1 < HALF)
                def _():
                    gather_start(cc + 1, ob)

    scatter_wait(HALF - 1, (HALF - 1) % NBUF)
    plsc.subcore_barrier()
    pltpu.sync_copy(a_sh.at[pl.ds(s * ROWS_PER_TILE, ROWS_PER_TILE)],
                    out_hbm.at[c, pl.ds(s * ROWS_PER_TILE, ROWS_PER_TILE)])


def kernel(x, edge_index, W_emb, b_emb, W_g, b_g, W_ih, W_hh, b_ih, b_hh,
           W_out, b_out):
    n_extra = E_PAD - N_EDGES
    pad_iota = jnp.arange(n_extra, dtype=jnp.int32)
    src = jnp.concatenate(
        [edge_index[0], pad_iota % N_NODES]
    ).reshape(NW, TILE_CHUNKS, CHUNK)
    dst = jnp.concatenate(
        [edge_index[1], N_NODES + pad_iota % (N_PAD - N_NODES)]
    ).reshape(NW, TILE_CHUNKS, CHUNK)
    zeros = jnp.zeros((N_PAD, HID), _f32)

    wembt = W_emb.T
    wgt = W_g.T
    wiht = W_ih.T
    whht = W_hh.T
    woutt = W_out.T
    bemb = b_emb.reshape(1, HID)
    bg = b_g.reshape(1, HID)
    bih = b_ih.reshape(1, 3 * HID)
    bhh = b_hh.reshape(1, 3 * HID)
    bout = b_out.reshape(1, HID)

    h, m = _embed_call(x, wembt, bemb, wgt, bg)
    for step in range(N_STEPS):
        parts = _msg_pass(m, src, dst, zeros)
        if step < N_STEPS - 1:
            h, m = _gru_mid_call(parts, h, wiht, bih, whht, bhh, wgt, bg)
        else:
            return _gru_last_call(parts, h, wiht, bih, whht, bhh, woutt, bout)


# speedup vs baseline: 1.1626x; 1.1626x over previous
import functools

import jax
import jax.numpy as jnp
from jax import lax
from jax.experimental import pallas as pl
from jax.experimental.pallas import tpu as pltpu
from jax.experimental.pallas import tpu_sc as plsc

N_NODES = 10000
N_EDGES = 320000
HID = 128
N_STEPS = 3

ROW_BLK = 1000
GRID = N_NODES // ROW_BLK

NC = 2
NS = 16
NW = NC * NS
CHUNK = 128
TILE_CHUNKS = 80
HALF = TILE_CHUNKS // 2
E_PAD = NW * TILE_CHUNKS * CHUNK
NBUF = 2
N_PAD = 10240
ROWS_PER_TILE = N_PAD // NS

def _dot(a, b):
    return jnp.dot(a.astype(jnp.bfloat16), b.astype(jnp.bfloat16),
                   preferred_element_type=jnp.float32)


def _embed_body(x_ref, wembt, bemb, wgt, bg, h_ref, m_ref):
    x = x_ref[...]
    h = _dot(x, wembt[...]) + bemb[...]
    h_ref[...] = h
    m_ref[...] = _dot(h, wgt[...]) + bg[...]


def _gru_core(a0_ref, a1_ref, h_ref, wiht, bih, whht, bhh):
    a = a0_ref[0] + a1_ref[0]
    h = h_ref[...]
    gi = _dot(a, wiht[...]) + bih[...]
    gh = _dot(h, whht[...]) + bhh[...]
    r = jax.nn.sigmoid(gi[:, :HID] + gh[:, :HID])
    z = jax.nn.sigmoid(gi[:, HID:2 * HID] + gh[:, HID:2 * HID])
    n = jnp.tanh(gi[:, 2 * HID:] + r * gh[:, 2 * HID:])
    return (1.0 - z) * n + z * h


def _gru_mid_body(a0_ref, a1_ref, h_ref, wiht, bih, whht, bhh, wgt, bg,
                  ho_ref, mo_ref):
    hn = _gru_core(a0_ref, a1_ref, h_ref, wiht, bih, whht, bhh)
    ho_ref[...] = hn
    mo_ref[...] = _dot(hn, wgt[...]) + bg[...]


def _gru_last_body(a0_ref, a1_ref, h_ref, wiht, bih, whht, bhh, woutt, bout,
                   out_ref):
    hn = _gru_core(a0_ref, a1_ref, h_ref, wiht, bih, whht, bhh)
    out_ref[...] = jnp.tanh(_dot(hn, woutt[...]) + bout[...])


def _row_spec(cols):
    return pl.BlockSpec((ROW_BLK, cols), lambda i: (i, 0))


def _full_spec(r, c):
    return pl.BlockSpec((r, c), lambda i: (0, 0))


def _part_spec(which):
    return pl.BlockSpec((1, ROW_BLK, HID), lambda i, w=which: (w, i, 0))


_f32 = jnp.float32


def _embed_call(x, wembt, bemb, wgt, bg):
    return pl.pallas_call(
        _embed_body,
        grid=(GRID,),
        in_specs=[
            _row_spec(HID),
            _full_spec(HID, HID), _full_spec(1, HID),
            _full_spec(HID, HID), _full_spec(1, HID),
        ],
        out_specs=[_row_spec(HID), _row_spec(HID)],
        out_shape=[
            jax.ShapeDtypeStruct((N_NODES, HID), _f32),
            jax.ShapeDtypeStruct((N_NODES, HID), _f32),
        ],
    )(x, wembt, bemb, wgt, bg)


def _gru_mid_call(parts, h, wiht, bih, whht, bhh, wgt, bg):
    return pl.pallas_call(
        _gru_mid_body,
        grid=(GRID,),
        in_specs=[
            _part_spec(0), _part_spec(1),
            _row_spec(HID),
            _full_spec(HID, 3 * HID), _full_spec(1, 3 * HID),
            _full_spec(HID, 3 * HID), _full_spec(1, 3 * HID),
            _full_spec(HID, HID), _full_spec(1, HID),
        ],
        out_specs=[_row_spec(HID), _row_spec(HID)],
        out_shape=[
            jax.ShapeDtypeStruct((N_NODES, HID), _f32),
            jax.ShapeDtypeStruct((N_NODES, HID), _f32),
        ],
    )(parts, parts, h, wiht, bih, whht, bhh, wgt, bg)


def _gru_last_call(parts, h, wiht, bih, whht, bhh, woutt, bout):
    return pl.pallas_call(
        _gru_last_body,
        grid=(GRID,),
        in_specs=[
            _part_spec(0), _part_spec(1),
            _row_spec(HID),
            _full_spec(HID, 3 * HID), _full_spec(1, 3 * HID),
            _full_spec(HID, 3 * HID), _full_spec(1, 3 * HID),
            _full_spec(HID, HID), _full_spec(1, HID),
        ],
        out_specs=_row_spec(HID),
        out_shape=jax.ShapeDtypeStruct((N_NODES, HID), _f32),
    )(parts, parts, h, wiht, bih, whht, bhh, woutt, bout)


@functools.partial(
    pl.kernel,
    out_type=jax.ShapeDtypeStruct((NC, N_PAD, HID), _f32),
    mesh=plsc.VectorSubcoreMesh(core_axis_name="c", subcore_axis_name="s"),
    scratch_types=[
        pltpu.VMEM((HALF, CHUNK), jnp.int32),
        pltpu.VMEM((HALF, CHUNK), jnp.int32),
        pltpu.VMEM((NBUF, CHUNK, HID), _f32),
        pltpu.VMEM_SHARED((N_PAD, HID), _f32),
        pltpu.SemaphoreType.DMA((NBUF,)),
    ],
)
def _msg_pass(m_hbm, srcs_hbm, dsts_hbm, zeros_hbm, out_hbm,
              src_t, dst_t, rows_v, a_sh, gsem):
    c = lax.axis_index("c")
    s = lax.axis_index("s")
    wid = c * NS + s

    def gather_start(cc, b):
        pltpu.async_copy(m_hbm.at[src_t.at[cc]], rows_v.at[b], gsem.at[b])

    def gather_wait(cc, b):
        pltpu.make_async_copy(m_hbm.at[src_t.at[cc]], rows_v.at[b],
                              gsem.at[b]).wait()

    pltpu.sync_copy(srcs_hbm.at[wid, pl.ds(0, HALF)], src_t)
    pltpu.sync_copy(dsts_hbm.at[wid, pl.ds(0, HALF)], dst_t)
    for b in range(NBUF):
        gather_start(b, b)
    pltpu.sync_copy(zeros_hbm.at[pl.ds(s * ROWS_PER_TILE, ROWS_PER_TILE)],
                    a_sh.at[pl.ds(s * ROWS_PER_TILE, ROWS_PER_TILE)])
    plsc.subcore_barrier()

    for hh in range(TILE_CHUNKS // HALF):
        if hh > 0:
            pltpu.sync_copy(srcs_hbm.at[wid, pl.ds(hh * HALF, HALF)], src_t)
            pltpu.sync_copy(dsts_hbm.at[wid, pl.ds(hh * HALF, HALF)], dst_t)
            for b in range(NBUF):
                gather_start(b, b)

        @pl.loop(0, HALF, step=NBUF)
        def _(t):
            for b in range(NBUF):
                cc = t + b
                gather_wait(cc, b)
                pltpu.sync_copy(rows_v.at[b], a_sh.at[dst_t.at[cc]],
                                add=True)

                @pl.when(cc + NBUF < HALF)
                def _():
                    gather_start(cc + NBUF, b)

    plsc.subcore_barrier()
    pltpu.sync_copy(a_sh.at[pl.ds(s * ROWS_PER_TILE, ROWS_PER_TILE)],
                    out_hbm.at[c, pl.ds(s * ROWS_PER_TILE, ROWS_PER_TILE)])


def kernel(x, edge_index, W_emb, b_emb, W_g, b_g, W_ih, W_hh, b_ih, b_hh,
           W_out, b_out):
    n_extra = E_PAD - N_EDGES
    pad_iota = jnp.arange(n_extra, dtype=jnp.int32)
    src = jnp.concatenate(
        [edge_index[0], pad_iota % N_NODES]
    ).reshape(NW, TILE_CHUNKS, CHUNK)
    dst = jnp.concatenate(
        [edge_index[1], N_NODES + pad_iota % (N_PAD - N_NODES)]
    ).reshape(NW, TILE_CHUNKS, CHUNK)
    zeros = jnp.zeros((N_PAD, HID), _f32)

    wembt = W_emb.T
    wgt = W_g.T
    wiht = W_ih.T
    whht = W_hh.T
    woutt = W_out.T
    bemb = b_emb.reshape(1, HID)
    bg = b_g.reshape(1, HID)
    bih = b_ih.reshape(1, 3 * HID)
    bhh = b_hh.reshape(1, 3 * HID)
    bout = b_out.reshape(1, HID)

    h, m = _embed_call(x, wembt, bemb, wgt, bg)
    for step in range(N_STEPS):
        parts = _msg_pass(m, src, dst, zeros)
        if step < N_STEPS - 1:
            h, m = _gru_mid_call(parts, h, wiht, bih, whht, bhh, wgt, bg)
        else:
            return _gru_last_call(parts, h, wiht, bih, whht, bhh, woutt, bout)


# speedup vs baseline: 1.1906x; 1.0241x over previous
import functools

import jax
import jax.numpy as jnp
from jax import lax
from jax.experimental import pallas as pl
from jax.experimental.pallas import tpu as pltpu
from jax.experimental.pallas import tpu_sc as plsc

N_NODES = 10000
N_EDGES = 320000
HID = 128
N_STEPS = 3

ROW_BLK = 2000
GRID = N_NODES // ROW_BLK

NC = 2
NS = 16
NW = NC * NS
CHUNK = 128
TILE_CHUNKS = 80
HALF = TILE_CHUNKS // 2
E_PAD = NW * TILE_CHUNKS * CHUNK
NBUF = 2
N_PAD = 10240
ROWS_PER_TILE = N_PAD // NS

def _dot(a, b):
    return jnp.dot(a.astype(jnp.bfloat16), b.astype(jnp.bfloat16),
                   preferred_element_type=jnp.float32)


def _embed_body(x_ref, wembt, bemb, wgt, bg, h_ref, m_ref):
    x = x_ref[...]
    h = _dot(x, wembt[...]) + bemb[...]
    h_ref[...] = h
    m_ref[...] = _dot(h, wgt[...]) + bg[...]


def _gru_core(a0_ref, a1_ref, h_ref, wiht, bih, whht, bhh):
    a = a0_ref[0] + a1_ref[0]
    h = h_ref[...]
    gi = _dot(a, wiht[...]) + bih[...]
    gh = _dot(h, whht[...]) + bhh[...]
    r = jax.nn.sigmoid(gi[:, :HID] + gh[:, :HID])
    z = jax.nn.sigmoid(gi[:, HID:2 * HID] + gh[:, HID:2 * HID])
    n = jnp.tanh(gi[:, 2 * HID:] + r * gh[:, 2 * HID:])
    return (1.0 - z) * n + z * h


def _gru_mid_body(a0_ref, a1_ref, h_ref, wiht, bih, whht, bhh, wgt, bg,
                  ho_ref, mo_ref):
    hn = _gru_core(a0_ref, a1_ref, h_ref, wiht, bih, whht, bhh)
    ho_ref[...] = hn
    mo_ref[...] = _dot(hn, wgt[...]) + bg[...]


def _gru_last_body(a0_ref, a1_ref, h_ref, wiht, bih, whht, bhh, woutt, bout,
                   out_ref):
    hn = _gru_core(a0_ref, a1_ref, h_ref, wiht, bih, whht, bhh)
    out_ref[...] = jnp.tanh(_dot(hn, woutt[...]) + bout[...])


def _row_spec(cols):
    return pl.BlockSpec((ROW_BLK, cols), lambda i: (i, 0))


def _full_spec(r, c):
    return pl.BlockSpec((r, c), lambda i: (0, 0))


def _part_spec(which):
    return pl.BlockSpec((1, ROW_BLK, HID), lambda i, w=which: (w, i, 0))


_f32 = jnp.float32


def _embed_call(x, wembt, bemb, wgt, bg):
    return pl.pallas_call(
        _embed_body,
        grid=(GRID,),
        in_specs=[
            _row_spec(HID),
            _full_spec(HID, HID), _full_spec(1, HID),
            _full_spec(HID, HID), _full_spec(1, HID),
        ],
        out_specs=[_row_spec(HID), _row_spec(HID)],
        out_shape=[
            jax.ShapeDtypeStruct((N_NODES, HID), _f32),
            jax.ShapeDtypeStruct((N_NODES, HID), _f32),
        ],
    )(x, wembt, bemb, wgt, bg)


def _gru_mid_call(parts, h, wiht, bih, whht, bhh, wgt, bg):
    return pl.pallas_call(
        _gru_mid_body,
        grid=(GRID,),
        in_specs=[
            _part_spec(0), _part_spec(1),
            _row_spec(HID),
            _full_spec(HID, 3 * HID), _full_spec(1, 3 * HID),
            _full_spec(HID, 3 * HID), _full_spec(1, 3 * HID),
            _full_spec(HID, HID), _full_spec(1, HID),
        ],
        out_specs=[_row_spec(HID), _row_spec(HID)],
        out_shape=[
            jax.ShapeDtypeStruct((N_NODES, HID), _f32),
            jax.ShapeDtypeStruct((N_NODES, HID), _f32),
        ],
    )(parts, parts, h, wiht, bih, whht, bhh, wgt, bg)


def _gru_last_call(parts, h, wiht, bih, whht, bhh, woutt, bout):
    return pl.pallas_call(
        _gru_last_body,
        grid=(GRID,),
        in_specs=[
            _part_spec(0), _part_spec(1),
            _row_spec(HID),
            _full_spec(HID, 3 * HID), _full_spec(1, 3 * HID),
            _full_spec(HID, 3 * HID), _full_spec(1, 3 * HID),
            _full_spec(HID, HID), _full_spec(1, HID),
        ],
        out_specs=_row_spec(HID),
        out_shape=jax.ShapeDtypeStruct((N_NODES, HID), _f32),
    )(parts, parts, h, wiht, bih, whht, bhh, woutt, bout)


@functools.partial(
    pl.kernel,
    out_type=jax.ShapeDtypeStruct((NC, N_PAD, HID), _f32),
    mesh=plsc.VectorSubcoreMesh(core_axis_name="c", subcore_axis_name="s"),
    scratch_types=[
        pltpu.VMEM((HALF, CHUNK), jnp.int32),
        pltpu.VMEM((HALF, CHUNK), jnp.int32),
        pltpu.VMEM((NBUF, CHUNK, HID), _f32),
        pltpu.VMEM_SHARED((N_PAD, HID), _f32),
        pltpu.SemaphoreType.DMA((NBUF,)),
    ],
)
def _msg_pass(m_hbm, srcs_hbm, dsts_hbm, zeros_hbm, out_hbm,
              src_t, dst_t, rows_v, a_sh, gsem):
    c = lax.axis_index("c")
    s = lax.axis_index("s")
    wid = c * NS + s

    def gather_start(cc, b):
        pltpu.async_copy(m_hbm.at[src_t.at[cc]], rows_v.at[b], gsem.at[b])

    def gather_wait(cc, b):
        pltpu.make_async_copy(m_hbm.at[src_t.at[cc]], rows_v.at[b],
                              gsem.at[b]).wait()

    pltpu.sync_copy(srcs_hbm.at[wid, pl.ds(0, HALF)], src_t)
    pltpu.sync_copy(dsts_hbm.at[wid, pl.ds(0, HALF)], dst_t)
    for b in range(NBUF):
        gather_start(b, b)
    pltpu.sync_copy(zeros_hbm.at[pl.ds(s * ROWS_PER_TILE, ROWS_PER_TILE)],
                    a_sh.at[pl.ds(s * ROWS_PER_TILE, ROWS_PER_TILE)])
    plsc.subcore_barrier()

    for hh in range(TILE_CHUNKS // HALF):
        if hh > 0:
            pltpu.sync_copy(srcs_hbm.at[wid, pl.ds(hh * HALF, HALF)], src_t)
            pltpu.sync_copy(dsts_hbm.at[wid, pl.ds(hh * HALF, HALF)], dst_t)
            for b in range(NBUF):
                gather_start(b, b)

        @pl.loop(0, HALF, step=NBUF)
        def _(t):
            for b in range(NBUF):
                cc = t + b
                gather_wait(cc, b)
                pltpu.sync_copy(rows_v.at[b], a_sh.at[dst_t.at[cc]],
                                add=True)

                @pl.when(cc + NBUF < HALF)
                def _():
                    gather_start(cc + NBUF, b)

    plsc.subcore_barrier()
    pltpu.sync_copy(a_sh.at[pl.ds(s * ROWS_PER_TILE, ROWS_PER_TILE)],
                    out_hbm.at[c, pl.ds(s * ROWS_PER_TILE, ROWS_PER_TILE)])


def kernel(x, edge_index, W_emb, b_emb, W_g, b_g, W_ih, W_hh, b_ih, b_hh,
           W_out, b_out):
    n_extra = E_PAD - N_EDGES
    pad_iota = jnp.arange(n_extra, dtype=jnp.int32)
    src = jnp.concatenate(
        [edge_index[0], pad_iota % N_NODES]
    ).reshape(NW, TILE_CHUNKS, CHUNK)
    dst = jnp.concatenate(
        [edge_index[1], N_NODES + pad_iota % (N_PAD - N_NODES)]
    ).reshape(NW, TILE_CHUNKS, CHUNK)
    zeros = jnp.zeros((N_PAD, HID), _f32)

    wembt = W_emb.T
    wgt = W_g.T
    wiht = W_ih.T
    whht = W_hh.T
    woutt = W_out.T
    bemb = b_emb.reshape(1, HID)
    bg = b_g.reshape(1, HID)
    bih = b_ih.reshape(1, 3 * HID)
    bhh = b_hh.reshape(1, 3 * HID)
    bout = b_out.reshape(1, HID)

    h, m = _embed_call(x, wembt, bemb, wgt, bg)
    for step in range(N_STEPS):
        parts = _msg_pass(m, src, dst, zeros)
        if step < N_STEPS - 1:
            h, m = _gru_mid_call(parts, h, wiht, bih, whht, bhh, wgt, bg)
        else:
            return _gru_last_call(parts, h, wiht, bih, whht, bhh, woutt, bout)


# speedup vs baseline: 1.1989x; 1.0070x over previous
import functools

import jax
import jax.numpy as jnp
from jax import lax
from jax.experimental import pallas as pl
from jax.experimental.pallas import tpu as pltpu
from jax.experimental.pallas import tpu_sc as plsc

N_NODES = 10000
N_EDGES = 320000
HID = 128
N_STEPS = 3

ROW_BLK = 5000
GRID = N_NODES // ROW_BLK

NC = 2
NS = 16
NW = NC * NS
CHUNK = 128
TILE_CHUNKS = 80
HALF = TILE_CHUNKS // 2
E_PAD = NW * TILE_CHUNKS * CHUNK
NBUF = 2
N_PAD = 10240
ROWS_PER_TILE = N_PAD // NS

def _dot(a, b):
    return jnp.dot(a.astype(jnp.bfloat16), b.astype(jnp.bfloat16),
                   preferred_element_type=jnp.float32)


def _embed_body(x_ref, wembt, bemb, wgt, bg, h_ref, m_ref):
    x = x_ref[...]
    h = _dot(x, wembt[...]) + bemb[...]
    h_ref[...] = h
    m_ref[...] = _dot(h, wgt[...]) + bg[...]


def _gru_core(a0_ref, a1_ref, h_ref, wiht, bih, whht, bhh):
    a = a0_ref[0] + a1_ref[0]
    h = h_ref[...]
    gi = _dot(a, wiht[...]) + bih[...]
    gh = _dot(h, whht[...]) + bhh[...]
    r = jax.nn.sigmoid(gi[:, :HID] + gh[:, :HID])
    z = jax.nn.sigmoid(gi[:, HID:2 * HID] + gh[:, HID:2 * HID])
    n = jnp.tanh(gi[:, 2 * HID:] + r * gh[:, 2 * HID:])
    return (1.0 - z) * n + z * h


def _gru_mid_body(a0_ref, a1_ref, h_ref, wiht, bih, whht, bhh, wgt, bg,
                  ho_ref, mo_ref):
    hn = _gru_core(a0_ref, a1_ref, h_ref, wiht, bih, whht, bhh)
    ho_ref[...] = hn
    mo_ref[...] = _dot(hn, wgt[...]) + bg[...]


def _gru_last_body(a0_ref, a1_ref, h_ref, wiht, bih, whht, bhh, woutt, bout,
                   out_ref):
    hn = _gru_core(a0_ref, a1_ref, h_ref, wiht, bih, whht, bhh)
    out_ref[...] = jnp.tanh(_dot(hn, woutt[...]) + bout[...])


def _row_spec(cols):
    return pl.BlockSpec((ROW_BLK, cols), lambda i: (i, 0))


def _full_spec(r, c):
    return pl.BlockSpec((r, c), lambda i: (0, 0))


def _part_spec(which):
    return pl.BlockSpec((1, ROW_BLK, HID), lambda i, w=which: (w, i, 0))


_f32 = jnp.float32


def _embed_call(x, wembt, bemb, wgt, bg):
    return pl.pallas_call(
        _embed_body,
        grid=(GRID,),
        in_specs=[
            _row_spec(HID),
            _full_spec(HID, HID), _full_spec(1, HID),
            _full_spec(HID, HID), _full_spec(1, HID),
        ],
        out_specs=[_row_spec(HID), _row_spec(HID)],
        out_shape=[
            jax.ShapeDtypeStruct((N_NODES, HID), _f32),
            jax.ShapeDtypeStruct((N_NODES, HID), _f32),
        ],
    )(x, wembt, bemb, wgt, bg)


def _gru_mid_call(parts, h, wiht, bih, whht, bhh, wgt, bg):
    return pl.pallas_call(
        _gru_mid_body,
        grid=(GRID,),
        in_specs=[
            _part_spec(0), _part_spec(1),
            _row_spec(HID),
            _full_spec(HID, 3 * HID), _full_spec(1, 3 * HID),
            _full_spec(HID, 3 * HID), _full_spec(1, 3 * HID),
            _full_spec(HID, HID), _full_spec(1, HID),
        ],
        out_specs=[_row_spec(HID), _row_spec(HID)],
        out_shape=[
            jax.ShapeDtypeStruct((N_NODES, HID), _f32),
            jax.ShapeDtypeStruct((N_NODES, HID), _f32),
        ],
    )(parts, parts, h, wiht, bih, whht, bhh, wgt, bg)


def _gru_last_call(parts, h, wiht, bih, whht, bhh, woutt, bout):
    return pl.pallas_call(
        _gru_last_body,
        grid=(GRID,),
        in_specs=[
            _part_spec(0), _part_spec(1),
            _row_spec(HID),
            _full_spec(HID, 3 * HID), _full_spec(1, 3 * HID),
            _full_spec(HID, 3 * HID), _full_spec(1, 3 * HID),
            _full_spec(HID, HID), _full_spec(1, HID),
        ],
        out_specs=_row_spec(HID),
        out_shape=jax.ShapeDtypeStruct((N_NODES, HID), _f32),
    )(parts, parts, h, wiht, bih, whht, bhh, woutt, bout)


@functools.partial(
    pl.kernel,
    out_type=jax.ShapeDtypeStruct((NC, N_PAD, HID), _f32),
    mesh=plsc.VectorSubcoreMesh(core_axis_name="c", subcore_axis_name="s"),
    scratch_types=[
        pltpu.VMEM((HALF, CHUNK), jnp.int32),
        pltpu.VMEM((HALF, CHUNK), jnp.int32),
        pltpu.VMEM((NBUF, CHUNK, HID), _f32),
        pltpu.VMEM_SHARED((N_PAD, HID), _f32),
        pltpu.SemaphoreType.DMA((NBUF,)),
    ],
)
def _msg_pass(m_hbm, srcs_hbm, dsts_hbm, zeros_hbm, out_hbm,
              src_t, dst_t, rows_v, a_sh, gsem):
    c = lax.axis_index("c")
    s = lax.axis_index("s")
    wid = c * NS + s

    def gather_start(cc, b):
        pltpu.async_copy(m_hbm.at[src_t.at[cc]], rows_v.at[b], gsem.at[b])

    def gather_wait(cc, b):
        pltpu.make_async_copy(m_hbm.at[src_t.at[cc]], rows_v.at[b],
                              gsem.at[b]).wait()

    pltpu.sync_copy(srcs_hbm.at[wid, pl.ds(0, HALF)], src_t)
    pltpu.sync_copy(dsts_hbm.at[wid, pl.ds(0, HALF)], dst_t)
    for b in range(NBUF):
        gather_start(b, b)
    pltpu.sync_copy(zeros_hbm.at[pl.ds(s * ROWS_PER_TILE, ROWS_PER_TILE)],
                    a_sh.at[pl.ds(s * ROWS_PER_TILE, ROWS_PER_TILE)])
    plsc.subcore_barrier()

    @pl.loop(0, TILE_CHUNKS // HALF)
    def _(hh):
        @pl.when(hh > 0)
        def _():
            off = pl.multiple_of(hh * HALF, HALF)
            pltpu.sync_copy(srcs_hbm.at[wid, pl.ds(off, HALF)], src_t)
            pltpu.sync_copy(dsts_hbm.at[wid, pl.ds(off, HALF)], dst_t)
            for b in range(NBUF):
                gather_start(b, b)

        @pl.loop(0, HALF, step=NBUF)
        def _(t):
            for b in range(NBUF):
                cc = t + b
                gather_wait(cc, b)
                pltpu.sync_copy(rows_v.at[b], a_sh.at[dst_t.at[cc]],
                                add=True)

                @pl.when(cc + NBUF < HALF)
                def _():
                    gather_start(cc + NBUF, b)

    plsc.subcore_barrier()
    pltpu.sync_copy(a_sh.at[pl.ds(s * ROWS_PER_TILE, ROWS_PER_TILE)],
                    out_hbm.at[c, pl.ds(s * ROWS_PER_TILE, ROWS_PER_TILE)])


def kernel(x, edge_index, W_emb, b_emb, W_g, b_g, W_ih, W_hh, b_ih, b_hh,
           W_out, b_out):
    n_extra = E_PAD - N_EDGES
    pad_iota = jnp.arange(n_extra, dtype=jnp.int32)
    src = jnp.concatenate(
        [edge_index[0], pad_iota % N_NODES]
    ).reshape(NW, TILE_CHUNKS, CHUNK)
    dst = jnp.concatenate(
        [edge_index[1], N_NODES + pad_iota % (N_PAD - N_NODES)]
    ).reshape(NW, TILE_CHUNKS, CHUNK)
    zeros = jnp.zeros((N_PAD, HID), _f32)

    wembt = W_emb.T
    wgt = W_g.T
    wiht = W_ih.T
    whht = W_hh.T
    woutt = W_out.T
    bemb = b_emb.reshape(1, HID)
    bg = b_g.reshape(1, HID)
    bih = b_ih.reshape(1, 3 * HID)
    bhh = b_hh.reshape(1, 3 * HID)
    bout = b_out.reshape(1, HID)

    h, m = _embed_call(x, wembt, bemb, wgt, bg)
    for step in range(N_STEPS):
        parts = _msg_pass(m, src, dst, zeros)
        if step < N_STEPS - 1:
            h, m = _gru_mid_call(parts, h, wiht, bih, whht, bhh, wgt, bg)
        else:
            return _gru_last_call(parts, h, wiht, bih, whht, bhh, woutt, bout)


# speedup vs baseline: 1.2345x; 1.0297x over previous
import functools

import jax
import jax.numpy as jnp
from jax import lax
from jax.experimental import pallas as pl
from jax.experimental.pallas import tpu as pltpu
from jax.experimental.pallas import tpu_sc as plsc

N_NODES = 10000
N_EDGES = 320000
HID = 128
N_STEPS = 3

ROW_BLK = 5000
GRID = N_NODES // ROW_BLK

NC = 2
NS = 16
NW = NC * NS
CHUNK = 120
TILE_CHUNKS = 88
QC = 8
NQ = TILE_CHUNKS // QC
E_PAD = NW * TILE_CHUNKS * CHUNK
NBUF = 3
N_PAD = 10112
ROWS_PER_TILE = N_PAD // NS

def _dot(a, b):
    return jnp.dot(a.astype(jnp.bfloat16), b.astype(jnp.bfloat16),
                   preferred_element_type=jnp.float32)


def _embed_body(x_ref, wembt, bemb, wgt, bg, h_ref, m_ref):
    x = x_ref[...]
    h = _dot(x, wembt[...]) + bemb[...]
    h_ref[...] = h
    m_ref[...] = _dot(h, wgt[...]) + bg[...]


def _gru_core(a0_ref, a1_ref, h_ref, wiht, bih, whht, bhh):
    a = a0_ref[0] + a1_ref[0]
    h = h_ref[...]
    gi = _dot(a, wiht[...]) + bih[...]
    gh = _dot(h, whht[...]) + bhh[...]
    r = jax.nn.sigmoid(gi[:, :HID] + gh[:, :HID])
    z = jax.nn.sigmoid(gi[:, HID:2 * HID] + gh[:, HID:2 * HID])
    n = jnp.tanh(gi[:, 2 * HID:] + r * gh[:, 2 * HID:])
    return (1.0 - z) * n + z * h


def _gru_mid_body(a0_ref, a1_ref, h_ref, wiht, bih, whht, bhh, wgt, bg,
                  ho_ref, mo_ref):
    hn = _gru_core(a0_ref, a1_ref, h_ref, wiht, bih, whht, bhh)
    ho_ref[...] = hn
    mo_ref[...] = _dot(hn, wgt[...]) + bg[...]


def _gru_last_body(a0_ref, a1_ref, h_ref, wiht, bih, whht, bhh, woutt, bout,
                   out_ref):
    hn = _gru_core(a0_ref, a1_ref, h_ref, wiht, bih, whht, bhh)
    out_ref[...] = jnp.tanh(_dot(hn, woutt[...]) + bout[...])


def _row_spec(cols):
    return pl.BlockSpec((ROW_BLK, cols), lambda i: (i, 0))


def _full_spec(r, c):
    return pl.BlockSpec((r, c), lambda i: (0, 0))


def _part_spec(which):
    return pl.BlockSpec((1, ROW_BLK, HID), lambda i, w=which: (w, i, 0))


_f32 = jnp.float32


def _embed_call(x, wembt, bemb, wgt, bg):
    return pl.pallas_call(
        _embed_body,
        grid=(GRID,),
        in_specs=[
            _row_spec(HID),
            _full_spec(HID, HID), _full_spec(1, HID),
            _full_spec(HID, HID), _full_spec(1, HID),
        ],
        out_specs=[_row_spec(HID), _row_spec(HID)],
        out_shape=[
            jax.ShapeDtypeStruct((N_NODES, HID), _f32),
            jax.ShapeDtypeStruct((N_NODES, HID), _f32),
        ],
    )(x, wembt, bemb, wgt, bg)


def _gru_mid_call(parts, h, wiht, bih, whht, bhh, wgt, bg):
    return pl.pallas_call(
        _gru_mid_body,
        grid=(GRID,),
        in_specs=[
            _part_spec(0), _part_spec(1),
            _row_spec(HID),
            _full_spec(HID, 3 * HID), _full_spec(1, 3 * HID),
            _full_spec(HID, 3 * HID), _full_spec(1, 3 * HID),
            _full_spec(HID, HID), _full_spec(1, HID),
        ],
        out_specs=[_row_spec(HID), _row_spec(HID)],
        out_shape=[
            jax.ShapeDtypeStruct((N_NODES, HID), _f32),
            jax.ShapeDtypeStruct((N_NODES, HID), _f32),
        ],
    )(parts, parts, h, wiht, bih, whht, bhh, wgt, bg)


def _gru_last_call(parts, h, wiht, bih, whht, bhh, woutt, bout):
    return pl.pallas_call(
        _gru_last_body,
        grid=(GRID,),
        in_specs=[
            _part_spec(0), _part_spec(1),
            _row_spec(HID),
            _full_spec(HID, 3 * HID), _full_spec(1, 3 * HID),
            _full_spec(HID, 3 * HID), _full_spec(1, 3 * HID),
            _full_spec(HID, HID), _full_spec(1, HID),
        ],
        out_specs=_row_spec(HID),
        out_shape=jax.ShapeDtypeStruct((N_NODES, HID), _f32),
    )(parts, parts, h, wiht, bih, whht, bhh, woutt, bout)


@functools.partial(
    pl.kernel,
    out_type=jax.ShapeDtypeStruct((NC, N_PAD, HID), _f32),
    mesh=plsc.VectorSubcoreMesh(core_axis_name="c", subcore_axis_name="s"),
    scratch_types=[
        pltpu.VMEM((2, QC, CHUNK), jnp.int32),
        pltpu.VMEM((2, QC, CHUNK), jnp.int32),
        pltpu.VMEM((NBUF, CHUNK, HID), _f32),
        pltpu.VMEM_SHARED((N_PAD, HID), _f32),
        pltpu.SemaphoreType.DMA((NBUF,)),
        pltpu.SemaphoreType.DMA((NBUF,)),
        pltpu.SemaphoreType.DMA((2,)),
        pltpu.SemaphoreType.DMA((2,)),
    ],
)
def _msg_pass(m_hbm, srcs_hbm, dsts_hbm, zeros_hbm, out_hbm,
              src_t, dst_t, rows_v, a_sh, gsem, ssem, sisem, disem):
    c = lax.axis_index("c")
    s = lax.axis_index("s")
    wid = c * NS + s

    def _idx_copies(q):
        qb = q % 2
        off = pl.multiple_of(q * QC, 8)
        return (
            pltpu.make_async_copy(srcs_hbm.at[wid, pl.ds(off, QC)],
                                  src_t.at[qb], sisem.at[qb]),
            pltpu.make_async_copy(dsts_hbm.at[wid, pl.ds(off, QC)],
                                  dst_t.at[qb], disem.at[qb]),
        )

    def idx_start(q):
        for cp in _idx_copies(q):
            cp.start()

    def idx_wait(q):
        for cp in _idx_copies(q):
            cp.wait()

    def _gather_copy(cc):
        b = cc % NBUF
        return pltpu.make_async_copy(m_hbm.at[src_t.at[(cc // QC) % 2,
                                                       cc % QC]],
                                     rows_v.at[b], gsem.at[b])

    def _scatter_copy(cc):
        b = cc % NBUF
        return pltpu.make_async_copy(rows_v.at[b],
                                     a_sh.at[dst_t.at[(cc // QC) % 2,
                                                      cc % QC]],
                                     ssem.at[b])

    idx_start(0)
    idx_start(1)
    idx_wait(0)
    _gather_copy(0).start()
    _gather_copy(1).start()
    pltpu.sync_copy(zeros_hbm.at[pl.ds(s * ROWS_PER_TILE, ROWS_PER_TILE)],
                    a_sh.at[pl.ds(s * ROWS_PER_TILE, ROWS_PER_TILE)])
    plsc.subcore_barrier()

    @pl.loop(0, NQ)
    def _(q):
        @pl.loop(0, QC)
        def _(tt):
            cc = q * QC + tt

            @pl.when(jnp.logical_and(tt == 1,
                                     jnp.logical_and(q >= 1, q + 1 < NQ)))
            def _():
                idx_start(q + 1)

            @pl.when(jnp.logical_and(tt == QC - 2, q + 1 < NQ))
            def _():
                idx_wait(q + 1)

            _gather_copy(cc).wait()
            _scatter_copy(cc).start(add=True)

            @pl.when(cc >= 1)
            def _():
                _scatter_copy(cc - 1).wait()

            @pl.when(cc + 2 < TILE_CHUNKS)
            def _():
                _gather_copy(cc + 2).start()

    _scatter_copy(TILE_CHUNKS - 1).wait()
    plsc.subcore_barrier()
    pltpu.sync_copy(a_sh.at[pl.ds(s * ROWS_PER_TILE, ROWS_PER_TILE)],
                    out_hbm.at[c, pl.ds(s * ROWS_PER_TILE, ROWS_PER_TILE)])


def kernel(x, edge_index, W_emb, b_emb, W_g, b_g, W_ih, W_hh, b_ih, b_hh,
           W_out, b_out):
    n_extra = E_PAD - N_EDGES
    pad_iota = jnp.arange(n_extra, dtype=jnp.int32)
    src = jnp.concatenate(
        [edge_index[0], pad_iota % N_NODES]
    ).reshape(NW, TILE_CHUNKS, CHUNK)
    dst = jnp.concatenate(
        [edge_index[1], N_NODES + pad_iota % (N_PAD - N_NODES)]
    ).reshape(NW, TILE_CHUNKS, CHUNK)
    zeros = jnp.zeros((N_PAD, HID), _f32)

    wembt = W_emb.T
    wgt = W_g.T
    wiht = W_ih.T
    whht = W_hh.T
    woutt = W_out.T
    bemb = b_emb.reshape(1, HID)
    bg = b_g.reshape(1, HID)
    bih = b_ih.reshape(1, 3 * HID)
    bhh = b_hh.reshape(1, 3 * HID)
    bout = b_out.reshape(1, HID)

    h, m = _embed_call(x, wembt, bemb, wgt, bg)
    for step in range(N_STEPS):
        parts = _msg_pass(m, src, dst, zeros)
        if step < N_STEPS - 1:
            h, m = _gru_mid_call(parts, h, wiht, bih, whht, bhh, wgt, bg)
        else:
            return _gru_last_call(parts, h, wiht, bih, whht, bhh, woutt, bout)
